# baseline (speedup 1.0000x reference)
import jax
import jax.numpy as jnp
from jax.experimental import pallas as pl
from jax.experimental.pallas import tpu as pltpu
from jax.experimental.pallas import tpu_sc as plsc

N = 50000
D_IN = 128
H = 64
G = 64
L = 3
E = 800000

R = 1024
NB = 49
NP = NB * R

NS = 16
C_HALF = 25000
DUMP = C_HALF
C_ACC = 25088
STRIPE = C_ACC // NS
LAST_ROWS = C_HALF - (NS - 1) * STRIPE

B = 128
CPG = 8
BG = B * CPG
EPT = 51200
EP = NS * EPT
NG = EPT // BG

_sc_mesh = plsc.VectorSubcoreMesh(core_axis_name="c", subcore_axis_name="s")


def _stripe_loop(s, fn):
    @pl.when(s < NS - 1)
    def _():
        for k in range(STRIPE // B):
            fn(k * B, B)
        if STRIPE % B:
            fn((STRIPE // B) * B, STRIPE % B)

    @pl.when(s == NS - 1)
    def _():
        for k in range(LAST_ROWS // B):
            fn(k * B, B)
        if LAST_ROWS % B:
            fn((LAST_ROWS // B) * B, LAST_ROWS % B)


def _sc_part_body(dst_hbm, src_hbm, deg_hbm, esrc_hbm, eldst_hbm, cnt_hbm,
                  acc, dst_g, src_g, idx_g, q_src, q_ldst, ones_v, stage_v,
                  cnt_v, ssem_a, ssem_b):
    c = jax.lax.axis_index("c")
    s = jax.lax.axis_index("s")
    base_node = c * C_HALF
    rowbase = (c * NS + s) * (EPT // B)

    def fill_ones(j, carry):
        ones_v[pl.ds(j * 16, 16)] = jnp.full((16,), 1.0, jnp.float32)
        return carry
    jax.lax.fori_loop(0, B // 16, fill_ones, 0)

    def fill_zero(j, carry):
        stage_v[pl.ds(j * 16, 16)] = jnp.zeros((16,), jnp.float32)
        return carry
    jax.lax.fori_loop(0, B // 16, fill_zero, 0)

    def zero_sub(off, ln):
        pltpu.sync_copy(stage_v.at[pl.ds(0, ln)],
                        acc.at[pl.ds(s * STRIPE + off, ln)])
    _stripe_loop(s, zero_sub)
    plsc.subcore_barrier()

    ebase = s * (EPT // B)
    ssem = (ssem_a, ssem_b)

    def scatter_wait(j):
        pltpu.make_async_copy(ones_v, acc.at[idx_g.at[j]], ssem[j % 2]).wait()

    def group(g, qoff):
        @pl.when(g > 0)
        def _():
            scatter_wait(CPG - 2)
            scatter_wait(CPG - 1)
        row = ebase + g * CPG
        pltpu.sync_copy(dst_hbm.at[pl.ds(row, CPG)], dst_g)
        pltpu.sync_copy(src_hbm.at[pl.ds(row, CPG)], src_g)
        for j in range(CPG):
            def grp(q, qo, j=j):
                d = dst_g[j, pl.ds(q * 16, 16)]
                local = d - base_node
                inb = (local >= 0) & (local < C_HALF)
                idx_g[j, pl.ds(q * 16, 16)] = jnp.where(inb, local, DUMP)
                inb_i = jnp.where(inb, 1, 0)
                incl = plsc.cumsum(inb_i)
                t = qo + (incl - inb_i)
                row_t = jax.lax.shift_right_logical(t, 7)
                col_t = jax.lax.bitwise_and(t, B - 1)
                sv = src_g[j, pl.ds(q * 16, 16)]
                plsc.store_scatter(q_src, [row_t, col_t], sv, mask=inb)
                plsc.store_scatter(q_ldst, [row_t, col_t], local, mask=inb)
                return qo + jnp.max(incl)
            qoff = jax.lax.fori_loop(0, B // 16, grp, qoff)
        for j in range(CPG):
            if j >= 2:
                scatter_wait(j - 2)
            pltpu.async_copy(ones_v, acc.at[idx_g.at[j]], ssem[j % 2],
                             add=True)
        return qoff
    qoff = jax.lax.fori_loop(0, NG, group, jnp.int32(0))
    scatter_wait(CPG - 2)
    scatter_wait(CPG - 1)

    pad_end = jax.lax.shift_left(
        jax.lax.shift_right_logical(qoff + (BG - 1), 10), 10)
    lane = jax.lax.iota(jnp.int32, 16)
    for k in range(BG // 16):
        t = qoff + k * 16 + lane
        pm = t < pad_end
        row_t = jax.lax.shift_right_logical(t, 7)
        col_t = jax.lax.bitwise_and(t, B - 1)
        plsc.store_scatter(q_src, [row_t, col_t],
                           jnp.zeros((16,), jnp.int32), mask=pm)
        plsc.store_scatter(q_ldst, [row_t, col_t],
                           jnp.full((16,), DUMP, jnp.int32), mask=pm)

    pltpu.sync_copy(q_src, esrc_hbm.at[pl.ds(rowbase, EPT // B)])
    pltpu.sync_copy(q_ldst, eldst_hbm.at[pl.ds(rowbase, EPT // B)])
    ngr = jax.lax.shift_right_logical(qoff + (BG - 1), 10)
    cnt_v[...] = jnp.full((16,), ngr, jnp.int32)
    pltpu.sync_copy(cnt_v, cnt_hbm.at[c * NS + s])

    plsc.subcore_barrier()

    def out_sub(off, ln):
        pltpu.sync_copy(acc.at[pl.ds(s * STRIPE + off, ln)],
                        stage_v.at[pl.ds(0, ln)])
        pltpu.sync_copy(stage_v.at[pl.ds(0, ln)],
                        deg_hbm.at[pl.ds(base_node + s * STRIPE + off, ln)])
    _stripe_loop(s, out_sub)


def _sc_part(dst_p, src_p):
    return pl.kernel(
        _sc_part_body,
        out_type=[
            jax.ShapeDtypeStruct((NP,), jnp.float32),
            jax.ShapeDtypeStruct((2 * NS * (EPT // B), B), jnp.int32),
            jax.ShapeDtypeStruct((2 * NS * (EPT // B), B), jnp.int32),
            jax.ShapeDtypeStruct((2 * NS, 16), jnp.int32),
        ],
        mesh=_sc_mesh,
        scratch_types=[
            pltpu.VMEM_SHARED((C_ACC,), jnp.float32),
            pltpu.VMEM((CPG, B), jnp.int32),
            pltpu.VMEM((CPG, B), jnp.int32),
            pltpu.VMEM((CPG, B), jnp.int32),
            pltpu.VMEM((EPT // B, B), jnp.int32),
            pltpu.VMEM((EPT // B, B), jnp.int32),
            pltpu.VMEM((B,), jnp.float32),
            pltpu.VMEM((B,), jnp.float32),
            pltpu.VMEM((16,), jnp.int32),
            pltpu.SemaphoreType.DMA,
            pltpu.SemaphoreType.DMA,
        ],
        compiler_params=pltpu.CompilerParams(needs_layout_passes=False),
    )(dst_p, src_p)


def _sc_agg_body(hwp_hbm, esrc_hbm, eldst_hbm, cnt_hbm, agg_hbm, acc,
                 src_g, ldst_g, rows_a, rows_b, rows_c, cnt_v,
                 gsem_a, gsem_b, gsem_c, ssem_a, ssem_b, ssem_c):
    c = jax.lax.axis_index("c")
    s = jax.lax.axis_index("s")
    base_node = c * C_HALF
    rowbase = (c * NS + s) * (EPT // B)
    rows = (rows_a, rows_b, rows_c)
    gsem = (gsem_a, gsem_b, gsem_c)
    ssem = (ssem_a, ssem_b, ssem_c)

    def init_sub(off, ln):
        pltpu.sync_copy(hwp_hbm.at[pl.ds(base_node + s * STRIPE + off, ln)],
                        rows_a.at[pl.ds(0, ln)])
        pltpu.sync_copy(rows_a.at[pl.ds(0, ln)],
                        acc.at[pl.ds(s * STRIPE + off, ln)])
    _stripe_loop(s, init_sub)

    pltpu.sync_copy(cnt_hbm.at[c * NS + s], cnt_v)
    ngr = jnp.max(cnt_v[...])
    plsc.subcore_barrier()

    def gather_start(j):
        return pltpu.async_copy(hwp_hbm.at[src_g.at[j]], rows[j % 3],
                                gsem[j % 3])

    def gather_wait(j):
        pltpu.make_async_copy(hwp_hbm.at[src_g.at[j]], rows[j % 3],
                              gsem[j % 3]).wait()

    def scatter_start(j):
        return pltpu.async_copy(rows[j % 3], acc.at[ldst_g.at[j]],
                                ssem[j % 3], add=True)

    def scatter_wait(j):
        pltpu.make_async_copy(rows[j % 3], acc.at[ldst_g.at[j]],
                              ssem[j % 3]).wait()

    def group(g, carry):
        @pl.when(g > 0)
        def _():
            scatter_wait(CPG - 3)
            scatter_wait(CPG - 2)
            scatter_wait(CPG - 1)
        row = rowbase + g * CPG
        pltpu.sync_copy(esrc_hbm.at[pl.ds(row, CPG)], src_g)
        pltpu.sync_copy(eldst_hbm.at[pl.ds(row, CPG)], ldst_g)
        gather_start(0)
        gather_start(1)
        for j in range(2, CPG):
            if j >= 3:
                scatter_wait(j - 3)
            gather_start(j)
            gather_wait(j - 2)
            scatter_start(j - 2)
        gather_wait(CPG - 2)
        scatter_start(CPG - 2)
        gather_wait(CPG - 1)
        scatter_start(CPG - 1)
        return carry
    jax.lax.fori_loop(0, ngr, group, 0)

    @pl.when(ngr > 0)
    def _():
        scatter_wait(CPG - 3)
        scatter_wait(CPG - 2)
        scatter_wait(CPG - 1)
    plsc.subcore_barrier()

    def out_sub(off, ln):
        pltpu.sync_copy(acc.at[pl.ds(s * STRIPE + off, ln)],
                        rows_a.at[pl.ds(0, ln)])
        pltpu.sync_copy(rows_a.at[pl.ds(0, ln)],
                        agg_hbm.at[pl.ds(base_node + s * STRIPE + off, ln)])
    _stripe_loop(s, out_sub)


def _sc_agg(hwp, esrc, eldst, cnts):
    return pl.kernel(
        _sc_agg_body,
        out_type=jax.ShapeDtypeStruct((NP, H), jnp.float32),
        mesh=_sc_mesh,
        scratch_types=[
            pltpu.VMEM_SHARED((C_ACC, H), jnp.float32),
            pltpu.VMEM((CPG, B), jnp.int32),
            pltpu.VMEM((CPG, B), jnp.int32),
            pltpu.VMEM((B, H), jnp.float32),
            pltpu.VMEM((B, H), jnp.float32),
            pltpu.VMEM((B, H), jnp.float32),
            pltpu.VMEM((16,), jnp.int32),
            pltpu.SemaphoreType.DMA,
            pltpu.SemaphoreType.DMA,
            pltpu.SemaphoreType.DMA,
            pltpu.SemaphoreType.DMA,
            pltpu.SemaphoreType.DMA,
            pltpu.SemaphoreType.DMA,
        ],
        compiler_params=pltpu.CompilerParams(use_tc_tiling_on_sc=False,
                                             needs_layout_passes=False),
    )(hwp, esrc, eldst, cnts)


_PREC = jax.lax.Precision.HIGHEST


def _dot(a, b, dims):
    return jax.lax.dot_general(a, b, dims, precision=_PREC,
                               preferred_element_type=jnp.float32)


def _dotT(a, b):
    return _dot(a, b, (((1,), (1,)), ((), ())))


def _row_mask(i):
    rowid = jax.lax.broadcasted_iota(jnp.int32, (R, 1), 0) + i * R
    return rowid < N


def _tc_in_body(x_ref, deg_ref, w_in_ref, b_in_ref, w0_ref,
                h_ref, hwp_ref, dis_ref):
    i = pl.program_id(0)
    m = _row_mask(i)
    h = jnp.where(m, _dotT(x_ref[...], w_in_ref[...]) + b_in_ref[...], 0.0)
    dis = jnp.where(m, 1.0 / jnp.sqrt(deg_ref[...] + 1.0), 0.0)
    h_ref[...] = h
    dis_ref[...] = dis
    hwp_ref[...] = dis * _dotT(h, w0_ref[...])


def _tc_in(x, deg2, w_in, b_in2, w0):
    return pl.pallas_call(
        _tc_in_body,
        grid=(NB,),
        in_specs=[
            pl.BlockSpec((R, D_IN), lambda i: (i, 0)),
            pl.BlockSpec((R, 1), lambda i: (i, 0)),
            pl.BlockSpec((H, D_IN), lambda i: (0, 0)),
            pl.BlockSpec((1, H), lambda i: (0, 0)),
            pl.BlockSpec((H, H), lambda i: (0, 0)),
        ],
        out_specs=[
            pl.BlockSpec((R, H), lambda i: (i, 0)),
            pl.BlockSpec((R, H), lambda i: (i, 0)),
            pl.BlockSpec((R, 1), lambda i: (i, 0)),
        ],
        out_shape=[
            jax.ShapeDtypeStruct((NP, H), jnp.float32),
            jax.ShapeDtypeStruct((NP, H), jnp.float32),
            jax.ShapeDtypeStruct((NP, 1), jnp.float32),
        ],
    )(x, deg2, w_in, b_in2, w0)


def _norm_stats(i, phase, agg_ref, dis_ref, batch_ref, b_ref, w_ref,
                bg_ref, a_ref, sz_ref, sz2_ref, cnt_ref, scale_ref, shift_ref):
    m = _row_mask(i % NB)
    z = jnp.where(m, dis_ref[...] * agg_ref[...] + b_ref[...], 0.0)
    gid = jax.lax.broadcasted_iota(jnp.int32, (R, G), 1)
    onehot = jnp.where(batch_ref[...] == gid, 1.0, 0.0)

    @pl.when(i == 0)
    def _():
        sz_ref[...] = jnp.zeros_like(sz_ref)
        sz2_ref[...] = jnp.zeros_like(sz2_ref)
        cnt_ref[...] = jnp.zeros_like(cnt_ref)

    @pl.when(phase == 0)
    def _():
        ones_col = jnp.where(m, 1.0, 0.0)
        colT = (((0,), (0,)), ((), ()))
        sz_ref[...] += _dot(onehot, z, colT)
        sz2_ref[...] += _dot(onehot, z * z, colT)
        cnt_ref[...] += _dot(onehot, ones_col, colT)

    @pl.when(i == NB - 1)
    def _():
        cnt = jnp.maximum(cnt_ref[...], 1.0)
        mean = sz_ref[...] / cnt
        m2 = sz2_ref[...] / cnt
        a = a_ref[...]
        var = m2 - (2.0 * a - a * a) * (mean * mean)
        std = jnp.sqrt(var + 1e-5)
        w = w_ref[...]
        scale_ref[...] = w / std
        shift_ref[...] = bg_ref[...] - w * a * mean / std
    return z, onehot


def _tc_layer_body(agg_ref, dis_ref, batch_ref, b_ref, w_ref, bg_ref, a_ref,
                   h_ref, wn_ref, hn_ref, hwpn_ref,
                   sz_ref, sz2_ref, cnt_ref, scale_ref, shift_ref):
    i = pl.program_id(0)
    phase = i // NB
    z, onehot = _norm_stats(i, phase, agg_ref, dis_ref, batch_ref, b_ref,
                            w_ref, bg_ref, a_ref, sz_ref, sz2_ref, cnt_ref,
                            scale_ref, shift_ref)

    @pl.when(phase == 1)
    def _():
        mm = (((1,), (0,)), ((), ()))
        sb = _dot(onehot, scale_ref[...], mm)
        hb = _dot(onehot, shift_ref[...], mm)
        r = jnp.maximum(sb * z + hb, 0.0)
        hn = h_ref[...] + r
        hn_ref[...] = hn
        hwpn_ref[...] = dis_ref[...] * _dotT(hn, wn_ref[...])


def _tc_layer(agg, dis, batch2, b_l, gw, gb, ga, h, wn):
    blk = lambda i: (i % NB, 0)
    cst = lambda i: (0, 0)
    return pl.pallas_call(
        _tc_layer_body,
        grid=(2 * NB,),
        in_specs=[
            pl.BlockSpec((R, H), blk),
            pl.BlockSpec((R, 1), blk),
            pl.BlockSpec((R, 1), blk),
            pl.BlockSpec((1, H), cst),
            pl.BlockSpec((1, H), cst),
            pl.BlockSpec((1, H), cst),
            pl.BlockSpec((1, H), cst),
            pl.BlockSpec((R, H), blk),
            pl.BlockSpec((H, H), cst),
        ],
        out_specs=[
            pl.BlockSpec((R, H), blk),
            pl.BlockSpec((R, H), blk),
        ],
        out_shape=[
            jax.ShapeDtypeStruct((NP, H), jnp.float32),
            jax.ShapeDtypeStruct((NP, H), jnp.float32),
        ],
        scratch_shapes=[
            pltpu.VMEM((G, H), jnp.float32),
            pltpu.VMEM((G, H), jnp.float32),
            pltpu.VMEM((G, 1), jnp.float32),
            pltpu.VMEM((G, H), jnp.float32),
            pltpu.VMEM((G, H), jnp.float32),
        ],
    )(agg, dis, batch2, b_l, gw, gb, ga, h, wn)


def _tc_last_body(agg_ref, dis_ref, batch_ref, b_ref, w_ref, bg_ref, a_ref,
                  h_ref, w1_ref, b1_ref, w2_ref, b2_ref, out_ref,
                  sz_ref, sz2_ref, cnt_ref, scale_ref, shift_ref,
                  pool_ref, pcnt_ref):
    i = pl.program_id(0)
    phase = i // NB
    z, onehot = _norm_stats(i, phase, agg_ref, dis_ref, batch_ref, b_ref,
                            w_ref, bg_ref, a_ref, sz_ref, sz2_ref, cnt_ref,
                            scale_ref, shift_ref)

    @pl.when(i == NB)
    def _():
        pool_ref[...] = jnp.zeros_like(pool_ref)
        pcnt_ref[...] = jnp.zeros_like(pcnt_ref)

    @pl.when(phase == 1)
    def _():
        m = _row_mask(i % NB)
        mm = (((1,), (0,)), ((), ()))
        sb = _dot(onehot, scale_ref[...], mm)
        hb = _dot(onehot, shift_ref[...], mm)
        r = jnp.maximum(sb * z + hb, 0.0)
        hn = h_ref[...] + r
        colT = (((0,), (0,)), ((), ()))
        pool_ref[...] += _dot(onehot, hn, colT)
        ones_col = jnp.where(m, 1.0, 0.0)
        pcnt_ref[...] += _dot(onehot, ones_col, colT)

    @pl.when(i == 2 * NB - 1)
    def _():
        cnt = jnp.maximum(pcnt_ref[...], 1.0)
        pooled = pool_ref[...] / cnt
        hid = jnp.maximum(_dotT(pooled, w1_ref[...]) + b1_ref[...], 0.0)
        out_ref[...] = _dotT(hid, w2_ref[...]) + b2_ref[...]


def _tc_last(agg, dis, batch2, b_l, gw, gb, ga, h, w1, b1r, w2, b2r):
    blk = lambda i: (i % NB, 0)
    cst = lambda i: (0, 0)
    return pl.pallas_call(
        _tc_last_body,
        grid=(2 * NB,),
        in_specs=[
            pl.BlockSpec((R, H), blk),
            pl.BlockSpec((R, 1), blk),
            pl.BlockSpec((R, 1), blk),
            pl.BlockSpec((1, H), cst),
            pl.BlockSpec((1, H), cst),
            pl.BlockSpec((1, H), cst),
            pl.BlockSpec((1, H), cst),
            pl.BlockSpec((R, H), blk),
            pl.BlockSpec((H, H), cst),
            pl.BlockSpec((1, H), cst),
            pl.BlockSpec((2, H), cst),
            pl.BlockSpec((1, 2), cst),
        ],
        out_specs=[
            pl.BlockSpec((G, 2), cst),
        ],
        out_shape=[
            jax.ShapeDtypeStruct((G, 2), jnp.float32),
        ],
        scratch_shapes=[
            pltpu.VMEM((G, H), jnp.float32),
            pltpu.VMEM((G, H), jnp.float32),
            pltpu.VMEM((G, 1), jnp.float32),
            pltpu.VMEM((G, H), jnp.float32),
            pltpu.VMEM((G, H), jnp.float32),
            pltpu.VMEM((G, H), jnp.float32),
            pltpu.VMEM((G, 1), jnp.float32),
        ],
    )(agg, dis, batch2, b_l, gw, gb, ga, h, w1, b1r, w2, b2r)


def kernel(x, edge_index, batch, W_in, b_in, conv_W, conv_b,
           gn_w, gn_b, gn_a, W1, b1, W2, b2):
    src = edge_index[0].astype(jnp.int32)
    dst = edge_index[1].astype(jnp.int32)
    src_p = jnp.pad(src, (0, EP - E), constant_values=0).reshape(EP // B, B)
    dst_p = jnp.pad(dst, (0, EP - E),
                    constant_values=N + 10000).reshape(EP // B, B)
    batch2 = jnp.pad(batch.astype(jnp.int32), (0, NP - N),
                     constant_values=G).reshape(NP, 1)

    deg, esrc, eldst, cnts = _sc_part(dst_p, src_p)
    h, hwp, dis = _tc_in(x, deg.reshape(NP, 1), W_in,
                         b_in.reshape(1, H), conv_W[0])
    out = None
    for l in range(L):
        agg = _sc_agg(hwp, esrc, eldst, cnts)
        gparams = (conv_b[l].reshape(1, H), gn_w[l].reshape(1, H),
                   gn_b[l].reshape(1, H), gn_a[l].reshape(1, H))
        if l < L - 1:
            h, hwp = _tc_layer(agg, dis, batch2, *gparams, h, conv_W[l + 1])
        else:
            (out,) = _tc_last(agg, dis, batch2, *gparams, h,
                              W1, b1.reshape(1, H), W2, b2.reshape(1, 2))
    return out


# speedup vs baseline: 1.0227x; 1.0227x over previous
import jax
import jax.numpy as jnp
from jax.experimental import pallas as pl
from jax.experimental.pallas import tpu as pltpu
from jax.experimental.pallas import tpu_sc as plsc

N = 50000
D_IN = 128
H = 64
G = 64
L = 3
E = 800000

R = 1024
NB = 49
NP = NB * R

NS = 16
C_HALF = 25000
DUMP = C_HALF
C_ACC = 25088
STRIPE = C_ACC // NS
LAST_ROWS = C_HALF - (NS - 1) * STRIPE

B = 128
CPG = 8
BG = B * CPG
EPT = 51200
EP = NS * EPT
NG = EPT // BG

_sc_mesh = plsc.VectorSubcoreMesh(core_axis_name="c", subcore_axis_name="s")


def _stripe_loop(s, fn):
    @pl.when(s < NS - 1)
    def _():
        for k in range(STRIPE // B):
            fn(k * B, B)
        if STRIPE % B:
            fn((STRIPE // B) * B, STRIPE % B)

    @pl.when(s == NS - 1)
    def _():
        for k in range(LAST_ROWS // B):
            fn(k * B, B)
        if LAST_ROWS % B:
            fn((LAST_ROWS // B) * B, LAST_ROWS % B)


def _sc_part_body(dst_hbm, src_hbm, deg_hbm, esrc_hbm, eldst_hbm, cnt_hbm,
                  acc, dst_g, src_g, idx_g, q_src, q_ldst, ones_v, stage_v,
                  cnt_v, ssem_a, ssem_b):
    c = jax.lax.axis_index("c")
    s = jax.lax.axis_index("s")
    base_node = c * C_HALF
    rowbase = (c * NS + s) * (EPT // B)

    def fill_ones(j, carry):
        ones_v[pl.ds(j * 16, 16)] = jnp.full((16,), 1.0, jnp.float32)
        return carry
    jax.lax.fori_loop(0, B // 16, fill_ones, 0)

    def fill_zero(j, carry):
        stage_v[pl.ds(j * 16, 16)] = jnp.zeros((16,), jnp.float32)
        return carry
    jax.lax.fori_loop(0, B // 16, fill_zero, 0)

    def zero_sub(off, ln):
        pltpu.sync_copy(stage_v.at[pl.ds(0, ln)],
                        acc.at[pl.ds(s * STRIPE + off, ln)])
    _stripe_loop(s, zero_sub)
    plsc.subcore_barrier()

    ebase = s * (EPT // B)
    ssem = (ssem_a, ssem_b)

    def scatter_wait(j):
        pltpu.make_async_copy(ones_v, acc.at[idx_g.at[j]], ssem[j % 2]).wait()

    def group(g, qoff):
        @pl.when(g > 0)
        def _():
            scatter_wait(CPG - 2)
            scatter_wait(CPG - 1)
        row = ebase + g * CPG
        pltpu.sync_copy(dst_hbm.at[pl.ds(row, CPG)], dst_g)
        pltpu.sync_copy(src_hbm.at[pl.ds(row, CPG)], src_g)
        for j in range(CPG):
            def grp(q, qo, j=j):
                d = dst_g[j, pl.ds(q * 16, 16)]
                local = d - base_node
                inb = (local >= 0) & (local < C_HALF)
                idx_g[j, pl.ds(q * 16, 16)] = jnp.where(inb, local, DUMP)
                inb_i = jnp.where(inb, 1, 0)
                incl = plsc.cumsum(inb_i)
                t = qo + (incl - inb_i)
                row_t = jax.lax.shift_right_logical(t, 7)
                col_t = jax.lax.bitwise_and(t, B - 1)
                sv = src_g[j, pl.ds(q * 16, 16)]
                plsc.store_scatter(q_src, [row_t, col_t], sv, mask=inb)
                plsc.store_scatter(q_ldst, [row_t, col_t], local, mask=inb)
                return qo + plsc.all_reduce_population_count(inb)
            qoff = jax.lax.fori_loop(0, B // 16, grp, qoff)
        for j in range(CPG):
            if j >= 2:
                scatter_wait(j - 2)
            pltpu.async_copy(ones_v, acc.at[idx_g.at[j]], ssem[j % 2],
                             add=True)
        return qoff
    qoff_v = jax.lax.fori_loop(0, NG, group, jnp.zeros((16,), jnp.int32))
    qoff = jnp.max(qoff_v)
    scatter_wait(CPG - 2)
    scatter_wait(CPG - 1)

    pad_end = jax.lax.shift_left(
        jax.lax.shift_right_logical(qoff + (BG - 1), 10), 10)
    lane = jax.lax.iota(jnp.int32, 16)
    for k in range(BG // 16):
        t = qoff + k * 16 + lane
        pm = t < pad_end
        row_t = jax.lax.shift_right_logical(t, 7)
        col_t = jax.lax.bitwise_and(t, B - 1)
        plsc.store_scatter(q_src, [row_t, col_t],
                           jnp.zeros((16,), jnp.int32), mask=pm)
        plsc.store_scatter(q_ldst, [row_t, col_t],
                           jnp.full((16,), DUMP, jnp.int32), mask=pm)

    pltpu.sync_copy(q_src, esrc_hbm.at[pl.ds(rowbase, EPT // B)])
    pltpu.sync_copy(q_ldst, eldst_hbm.at[pl.ds(rowbase, EPT // B)])
    ngr = jax.lax.shift_right_logical(qoff + (BG - 1), 10)
    cnt_v[...] = jnp.full((16,), ngr, jnp.int32)
    pltpu.sync_copy(cnt_v, cnt_hbm.at[c * NS + s])

    plsc.subcore_barrier()

    def out_sub(off, ln):
        pltpu.sync_copy(acc.at[pl.ds(s * STRIPE + off, ln)],
                        stage_v.at[pl.ds(0, ln)])
        pltpu.sync_copy(stage_v.at[pl.ds(0, ln)],
                        deg_hbm.at[pl.ds(base_node + s * STRIPE + off, ln)])
    _stripe_loop(s, out_sub)


def _sc_part(dst_p, src_p):
    return pl.kernel(
        _sc_part_body,
        out_type=[
            jax.ShapeDtypeStruct((NP,), jnp.float32),
            jax.ShapeDtypeStruct((2 * NS * (EPT // B), B), jnp.int32),
            jax.ShapeDtypeStruct((2 * NS * (EPT // B), B), jnp.int32),
            jax.ShapeDtypeStruct((2 * NS, 16), jnp.int32),
        ],
        mesh=_sc_mesh,
        scratch_types=[
            pltpu.VMEM_SHARED((C_ACC,), jnp.float32),
            pltpu.VMEM((CPG, B), jnp.int32),
            pltpu.VMEM((CPG, B), jnp.int32),
            pltpu.VMEM((CPG, B), jnp.int32),
            pltpu.VMEM((EPT // B, B), jnp.int32),
            pltpu.VMEM((EPT // B, B), jnp.int32),
            pltpu.VMEM((B,), jnp.float32),
            pltpu.VMEM((B,), jnp.float32),
            pltpu.VMEM((16,), jnp.int32),
            pltpu.SemaphoreType.DMA,
            pltpu.SemaphoreType.DMA,
        ],
        compiler_params=pltpu.CompilerParams(needs_layout_passes=False),
    )(dst_p, src_p)


def _sc_agg_body(hwp_hbm, esrc_hbm, eldst_hbm, cnt_hbm, agg_hbm, acc,
                 src_g, ldst_g, rows_a, rows_b, rows_c, cnt_v,
                 gsem_a, gsem_b, gsem_c, ssem_a, ssem_b, ssem_c):
    c = jax.lax.axis_index("c")
    s = jax.lax.axis_index("s")
    base_node = c * C_HALF
    rowbase = (c * NS + s) * (EPT // B)
    rows = (rows_a, rows_b, rows_c)
    gsem = (gsem_a, gsem_b, gsem_c)
    ssem = (ssem_a, ssem_b, ssem_c)

    def init_sub(off, ln):
        pltpu.sync_copy(hwp_hbm.at[pl.ds(base_node + s * STRIPE + off, ln)],
                        rows_a.at[pl.ds(0, ln)])
        pltpu.sync_copy(rows_a.at[pl.ds(0, ln)],
                        acc.at[pl.ds(s * STRIPE + off, ln)])
    _stripe_loop(s, init_sub)

    pltpu.sync_copy(cnt_hbm.at[c * NS + s], cnt_v)
    ngr = jnp.max(cnt_v[...])
    plsc.subcore_barrier()

    def gather_start(j):
        return pltpu.async_copy(hwp_hbm.at[src_g.at[j]], rows[j % 3],
                                gsem[j % 3])

    def gather_wait(j):
        pltpu.make_async_copy(hwp_hbm.at[src_g.at[j]], rows[j % 3],
                              gsem[j % 3]).wait()

    def scatter_start(j):
        return pltpu.async_copy(rows[j % 3], acc.at[ldst_g.at[j]],
                                ssem[j % 3], add=True)

    def scatter_wait(j):
        pltpu.make_async_copy(rows[j % 3], acc.at[ldst_g.at[j]],
                              ssem[j % 3]).wait()

    def group(g, carry):
        @pl.when(g > 0)
        def _():
            scatter_wait(CPG - 3)
            scatter_wait(CPG - 2)
            scatter_wait(CPG - 1)
        row = rowbase + g * CPG
        pltpu.sync_copy(esrc_hbm.at[pl.ds(row, CPG)], src_g)
        pltpu.sync_copy(eldst_hbm.at[pl.ds(row, CPG)], ldst_g)
        gather_start(0)
        gather_start(1)
        for j in range(2, CPG):
            if j >= 3:
                scatter_wait(j - 3)
            gather_start(j)
            gather_wait(j - 2)
            scatter_start(j - 2)
        gather_wait(CPG - 2)
        scatter_start(CPG - 2)
        gather_wait(CPG - 1)
        scatter_start(CPG - 1)
        return carry
    jax.lax.fori_loop(0, ngr, group, 0)

    @pl.when(ngr > 0)
    def _():
        scatter_wait(CPG - 3)
        scatter_wait(CPG - 2)
        scatter_wait(CPG - 1)
    plsc.subcore_barrier()

    def out_sub(off, ln):
        pltpu.sync_copy(acc.at[pl.ds(s * STRIPE + off, ln)],
                        rows_a.at[pl.ds(0, ln)])
        pltpu.sync_copy(rows_a.at[pl.ds(0, ln)],
                        agg_hbm.at[pl.ds(base_node + s * STRIPE + off, ln)])
    _stripe_loop(s, out_sub)


def _sc_agg(hwp, esrc, eldst, cnts):
    return pl.kernel(
        _sc_agg_body,
        out_type=jax.ShapeDtypeStruct((NP, H), jnp.float32),
        mesh=_sc_mesh,
        scratch_types=[
            pltpu.VMEM_SHARED((C_ACC, H), jnp.float32),
            pltpu.VMEM((CPG, B), jnp.int32),
            pltpu.VMEM((CPG, B), jnp.int32),
            pltpu.VMEM((B, H), jnp.float32),
            pltpu.VMEM((B, H), jnp.float32),
            pltpu.VMEM((B, H), jnp.float32),
            pltpu.VMEM((16,), jnp.int32),
            pltpu.SemaphoreType.DMA,
            pltpu.SemaphoreType.DMA,
            pltpu.SemaphoreType.DMA,
            pltpu.SemaphoreType.DMA,
            pltpu.SemaphoreType.DMA,
            pltpu.SemaphoreType.DMA,
        ],
        compiler_params=pltpu.CompilerParams(use_tc_tiling_on_sc=False,
                                             needs_layout_passes=False),
    )(hwp, esrc, eldst, cnts)


_PREC = jax.lax.Precision.HIGHEST


def _dot(a, b, dims):
    return jax.lax.dot_general(a, b, dims, precision=_PREC,
                               preferred_element_type=jnp.float32)


def _dotT(a, b):
    return _dot(a, b, (((1,), (1,)), ((), ())))


def _row_mask(i):
    rowid = jax.lax.broadcasted_iota(jnp.int32, (R, 1), 0) + i * R
    return rowid < N


def _tc_in_body(x_ref, deg_ref, w_in_ref, b_in_ref, w0_ref,
                h_ref, hwp_ref, dis_ref):
    i = pl.program_id(0)
    m = _row_mask(i)
    h = jnp.where(m, _dotT(x_ref[...], w_in_ref[...]) + b_in_ref[...], 0.0)
    dis = jnp.where(m, 1.0 / jnp.sqrt(deg_ref[...] + 1.0), 0.0)
    h_ref[...] = h
    dis_ref[...] = dis
    hwp_ref[...] = dis * _dotT(h, w0_ref[...])


def _tc_in(x, deg2, w_in, b_in2, w0):
    return pl.pallas_call(
        _tc_in_body,
        grid=(NB,),
        in_specs=[
            pl.BlockSpec((R, D_IN), lambda i: (i, 0)),
            pl.BlockSpec((R, 1), lambda i: (i, 0)),
            pl.BlockSpec((H, D_IN), lambda i: (0, 0)),
            pl.BlockSpec((1, H), lambda i: (0, 0)),
            pl.BlockSpec((H, H), lambda i: (0, 0)),
        ],
        out_specs=[
            pl.BlockSpec((R, H), lambda i: (i, 0)),
            pl.BlockSpec((R, H), lambda i: (i, 0)),
            pl.BlockSpec((R, 1), lambda i: (i, 0)),
        ],
        out_shape=[
            jax.ShapeDtypeStruct((NP, H), jnp.float32),
            jax.ShapeDtypeStruct((NP, H), jnp.float32),
            jax.ShapeDtypeStruct((NP, 1), jnp.float32),
        ],
    )(x, deg2, w_in, b_in2, w0)


def _tc_p1_body(agg_ref, dis_ref, batch_ref, b_ref, w_ref, bg_ref, a_ref,
                z_ref, scale_ref, shift_ref, sz_ref, sz2_ref, cnt_ref):
    i = pl.program_id(0)

    @pl.when(i == 0)
    def _():
        sz_ref[...] = jnp.zeros_like(sz_ref)
        sz2_ref[...] = jnp.zeros_like(sz2_ref)
        cnt_ref[...] = jnp.zeros_like(cnt_ref)

    m = _row_mask(i)
    z = jnp.where(m, dis_ref[...] * agg_ref[...] + b_ref[...], 0.0)
    z_ref[...] = z
    gid = jax.lax.broadcasted_iota(jnp.int32, (R, G), 1)
    onehot = jnp.where(batch_ref[...] == gid, 1.0, 0.0)
    ones_col = jnp.where(m, 1.0, 0.0)
    colT = (((0,), (0,)), ((), ()))
    sz_ref[...] += _dot(onehot, z, colT)
    sz2_ref[...] += _dot(onehot, z * z, colT)
    cnt_ref[...] += _dot(onehot, ones_col, colT)

    @pl.when(i == NB - 1)
    def _():
        cnt = jnp.maximum(cnt_ref[...], 1.0)
        mean = sz_ref[...] / cnt
        m2 = sz2_ref[...] / cnt
        a = a_ref[...]
        var = m2 - (2.0 * a - a * a) * (mean * mean)
        std = jnp.sqrt(var + 1e-5)
        w = w_ref[...]
        scale_ref[...] = w / std
        shift_ref[...] = bg_ref[...] - w * a * mean / std


def _tc_p1(agg, dis, batch2, b_l, gw, gb, ga):
    return pl.pallas_call(
        _tc_p1_body,
        grid=(NB,),
        in_specs=[
            pl.BlockSpec((R, H), lambda i: (i, 0)),
            pl.BlockSpec((R, 1), lambda i: (i, 0)),
            pl.BlockSpec((R, 1), lambda i: (i, 0)),
            pl.BlockSpec((1, H), lambda i: (0, 0)),
            pl.BlockSpec((1, H), lambda i: (0, 0)),
            pl.BlockSpec((1, H), lambda i: (0, 0)),
            pl.BlockSpec((1, H), lambda i: (0, 0)),
        ],
        out_specs=[
            pl.BlockSpec((R, H), lambda i: (i, 0)),
            pl.BlockSpec((G, H), lambda i: (0, 0)),
            pl.BlockSpec((G, H), lambda i: (0, 0)),
        ],
        out_shape=[
            jax.ShapeDtypeStruct((NP, H), jnp.float32),
            jax.ShapeDtypeStruct((G, H), jnp.float32),
            jax.ShapeDtypeStruct((G, H), jnp.float32),
        ],
        scratch_shapes=[
            pltpu.VMEM((G, H), jnp.float32),
            pltpu.VMEM((G, H), jnp.float32),
            pltpu.VMEM((G, 1), jnp.float32),
        ],
    )(agg, dis, batch2, b_l, gw, gb, ga)


def _tc_p2_body(z_ref, h_ref, dis_ref, batch_ref, scale_ref, shift_ref,
                wn_ref, hn_ref, hwpn_ref):
    gid = jax.lax.broadcasted_iota(jnp.int32, (R, G), 1)
    onehot = jnp.where(batch_ref[...] == gid, 1.0, 0.0)
    mm = (((1,), (0,)), ((), ()))
    sb = _dot(onehot, scale_ref[...], mm)
    hb = _dot(onehot, shift_ref[...], mm)
    r = jnp.maximum(sb * z_ref[...] + hb, 0.0)
    hn = h_ref[...] + r
    hn_ref[...] = hn
    hwpn_ref[...] = dis_ref[...] * _dotT(hn, wn_ref[...])


def _tc_p2(z, h, dis, batch2, scale, shift, wn):
    return pl.pallas_call(
        _tc_p2_body,
        grid=(NB,),
        in_specs=[
            pl.BlockSpec((R, H), lambda i: (i, 0)),
            pl.BlockSpec((R, H), lambda i: (i, 0)),
            pl.BlockSpec((R, 1), lambda i: (i, 0)),
            pl.BlockSpec((R, 1), lambda i: (i, 0)),
            pl.BlockSpec((G, H), lambda i: (0, 0)),
            pl.BlockSpec((G, H), lambda i: (0, 0)),
            pl.BlockSpec((H, H), lambda i: (0, 0)),
        ],
        out_specs=[
            pl.BlockSpec((R, H), lambda i: (i, 0)),
            pl.BlockSpec((R, H), lambda i: (i, 0)),
        ],
        out_shape=[
            jax.ShapeDtypeStruct((NP, H), jnp.float32),
            jax.ShapeDtypeStruct((NP, H), jnp.float32),
        ],
    )(z, h, dis, batch2, scale, shift, wn)


def _tc_p2f_body(z_ref, h_ref, batch_ref, scale_ref, shift_ref,
                 w1_ref, b1_ref, w2_ref, b2_ref,
                 out_ref, pool_ref, cnt_ref):
    i = pl.program_id(0)

    @pl.when(i == 0)
    def _():
        pool_ref[...] = jnp.zeros_like(pool_ref)
        cnt_ref[...] = jnp.zeros_like(cnt_ref)

    m = _row_mask(i)
    gid = jax.lax.broadcasted_iota(jnp.int32, (R, G), 1)
    onehot = jnp.where(batch_ref[...] == gid, 1.0, 0.0)
    mm = (((1,), (0,)), ((), ()))
    sb = _dot(onehot, scale_ref[...], mm)
    hb = _dot(onehot, shift_ref[...], mm)
    r = jnp.maximum(sb * z_ref[...] + hb, 0.0)
    hn = h_ref[...] + r
    colT = (((0,), (0,)), ((), ()))
    pool_ref[...] += _dot(onehot, hn, colT)
    ones_col = jnp.where(m, 1.0, 0.0)
    cnt_ref[...] += _dot(onehot, ones_col, colT)

    @pl.when(i == NB - 1)
    def _():
        cnt = jnp.maximum(cnt_ref[...], 1.0)
        pooled = pool_ref[...] / cnt
        hid = jnp.maximum(_dotT(pooled, w1_ref[...]) + b1_ref[...], 0.0)
        out_ref[...] = _dotT(hid, w2_ref[...]) + b2_ref[...]


def _tc_p2f(z, h, batch2, scale, shift, w1, b1r, w2, b2r):
    return pl.pallas_call(
        _tc_p2f_body,
        grid=(NB,),
        in_specs=[
            pl.BlockSpec((R, H), lambda i: (i, 0)),
            pl.BlockSpec((R, H), lambda i: (i, 0)),
            pl.BlockSpec((R, 1), lambda i: (i, 0)),
            pl.BlockSpec((G, H), lambda i: (0, 0)),
            pl.BlockSpec((G, H), lambda i: (0, 0)),
            pl.BlockSpec((H, H), lambda i: (0, 0)),
            pl.BlockSpec((1, H), lambda i: (0, 0)),
            pl.BlockSpec((2, H), lambda i: (0, 0)),
            pl.BlockSpec((1, 2), lambda i: (0, 0)),
        ],
        out_specs=[
            pl.BlockSpec((G, 2), lambda i: (0, 0)),
        ],
        out_shape=[
            jax.ShapeDtypeStruct((G, 2), jnp.float32),
        ],
        scratch_shapes=[
            pltpu.VMEM((G, H), jnp.float32),
            pltpu.VMEM((G, 1), jnp.float32),
        ],
    )(z, h, batch2, scale, shift, w1, b1r, w2, b2r)


def kernel(x, edge_index, batch, W_in, b_in, conv_W, conv_b,
           gn_w, gn_b, gn_a, W1, b1, W2, b2):
    src = edge_index[0].astype(jnp.int32)
    dst = edge_index[1].astype(jnp.int32)
    src_p = jnp.pad(src, (0, EP - E), constant_values=0).reshape(EP // B, B)
    dst_p = jnp.pad(dst, (0, EP - E),
                    constant_values=N + 10000).reshape(EP // B, B)
    batch2 = jnp.pad(batch.astype(jnp.int32), (0, NP - N),
                     constant_values=G).reshape(NP, 1)

    deg, esrc, eldst, cnts = _sc_part(dst_p, src_p)
    h, hwp, dis = _tc_in(x, deg.reshape(NP, 1), W_in,
                         b_in.reshape(1, H), conv_W[0])
    out = None
    for l in range(L):
        agg = _sc_agg(hwp, esrc, eldst, cnts)
        z, scale, shift = _tc_p1(agg, dis, batch2,
                                 conv_b[l].reshape(1, H),
                                 gn_w[l].reshape(1, H),
                                 gn_b[l].reshape(1, H),
                                 gn_a[l].reshape(1, H))
        if l < L - 1:
            h, hwp = _tc_p2(z, h, dis, batch2, scale, shift, conv_W[l + 1])
        else:
            (out,) = _tc_p2f(z, h, batch2, scale, shift,
                             W1, b1.reshape(1, H), W2, b2.reshape(1, 2))
    return out
